# baseline (speedup 1.0000x reference)
import functools

import jax
import jax.numpy as jnp
from jax import lax
from jax.experimental import pallas as pl
from jax.experimental.pallas import tpu as pltpu
from jax.experimental.pallas import tpu_sc as plsc

N_NODES = 10000
N_EDGES = 320000
D_FEAT = 128
NPAD = 10240
NW = 32
CHUNK = 128
NCHUNKS = N_EDGES // CHUNK
BASE_CH = NCHUNKS // NW
EXTRA = NCHUNKS - BASE_CH * NW
NPT = NPAD // 16
DROWS = NPAD // 128


def _make_sc_agg(with_deg, NSLOT):
    mesh = plsc.VectorSubcoreMesh(core_axis_name="c", subcore_axis_name="s")
    out_type = [jax.ShapeDtypeStruct((2, N_NODES, D_FEAT), jnp.float32)]
    NIDX = 2 * NSLOT
    scratch = [
        [pltpu.VMEM((CHUNK,), jnp.int32) for _ in range(NIDX)],
        [pltpu.VMEM((CHUNK,), jnp.int32) for _ in range(NIDX)],
        [pltpu.VMEM((CHUNK, D_FEAT), jnp.float32) for _ in range(NSLOT)],
        pltpu.VMEM_SHARED((N_NODES, D_FEAT), jnp.float32),
        [pltpu.SemaphoreType.DMA for _ in range(NIDX)],
        [pltpu.SemaphoreType.DMA for _ in range(NSLOT)],
        [pltpu.SemaphoreType.DMA for _ in range(NSLOT)],
    ]
    if with_deg:
        out_type.append(jax.ShapeDtypeStruct((2, DROWS, D_FEAT), jnp.float32))
        scratch += [
            pltpu.VMEM((DROWS, D_FEAT), jnp.float32),
            pltpu.VMEM((DROWS,), jnp.int32),
            pltpu.VMEM_SHARED((DROWS, D_FEAT), jnp.float32),
        ]

    @functools.partial(
        pl.kernel, mesh=mesh, out_type=out_type, scratch_types=scratch,
        compiler_params=pltpu.CompilerParams(needs_layout_passes=False))
    def agg(xp_hbm, src_hbm, dst_hbm, *rest):
        if with_deg:
            (out_hbm, deg_hbm, src_s, dst_s, rows_s, acc,
             is_s, gs_s, ss_s, deg_v, iota_v, deg_acc) = rest
        else:
            (out_hbm, src_s, dst_s, rows_s, acc, is_s, gs_s, ss_s) = rest
        rows_v = rows_s[0]
        cid = lax.axis_index("c")
        sid = lax.axis_index("s")
        wid = sid * 2 + cid

        def zrow(i, carry):
            for j in range(D_FEAT // 16):
                rows_v[i, pl.ds(j * 16, 16)] = jnp.zeros((16,), jnp.float32)
            return carry
        lax.fori_loop(0, CHUNK, zrow, 0)
        r0 = pl.multiple_of(sid * 624, 8)

        @pl.when(sid < 15)
        def _():
            for m in range(4):
                pltpu.sync_copy(rows_v.at[pl.ds(0, CHUNK)],
                                acc.at[pl.ds(r0 + m * CHUNK, CHUNK)])
            pltpu.sync_copy(rows_v.at[pl.ds(0, 112)],
                            acc.at[pl.ds(r0 + 512, 112)])

        @pl.when(sid == 15)
        def _():
            for m in range(5):
                pltpu.sync_copy(rows_v.at[pl.ds(0, CHUNK)],
                                acc.at[pl.ds(9360 + m * CHUNK, CHUNK)])
        if with_deg:
            def zdeg(i, carry):
                for j in range(D_FEAT // 16):
                    deg_v[i, pl.ds(j * 16, 16)] = jnp.zeros((16,), jnp.float32)
                return carry
            lax.fori_loop(0, DROWS, zdeg, 0)
            for m in range(DROWS // 16):
                iota_v[pl.ds(m * 16, 16)] = (
                    lax.iota(jnp.int32, 16) + m * 16)

            @pl.when(sid == 0)
            def _():
                pltpu.sync_copy(rows_v.at[pl.ds(0, DROWS)], deg_acc)
        plsc.subcore_barrier()

        cstart = wid * BASE_CH + jnp.minimum(wid, EXTRA)
        nch = jnp.where(wid < EXTRA, BASE_CH + 1, BASE_CH)

        def idx_slices(c):
            eb = pl.multiple_of((cstart + c) * CHUNK, CHUNK)
            return src_hbm.at[pl.ds(eb, CHUNK)], dst_hbm.at[pl.ds(eb, CHUNK)]

        def idx_start(c, s):
            sr, dr = idx_slices(c)
            pltpu.async_copy(sr, src_s[s], is_s[s])
            pltpu.async_copy(dr, dst_s[s], is_s[s])

        def idx_wait(c, s):
            sr, dr = idx_slices(c)
            pltpu.make_async_copy(sr, src_s[s], is_s[s]).wait()
            pltpu.make_async_copy(dr, dst_s[s], is_s[s]).wait()

        def deg_update(dstx):
            if with_deg:
                for j in range(CHUNK // 16):
                    d16 = dstx[pl.ds(j * 16, 16)]
                    cnt, mlast = plsc.scan_count(d16)
                    plsc.addupdate_scatter(
                        deg_v,
                        [lax.shift_right_logical(d16, 7),
                         jnp.bitwise_and(d16, 127)],
                        cnt.astype(jnp.float32), mask=mlast)

        def gather_start(ii, rr):
            pltpu.async_copy(xp_hbm.at[src_s[ii]], rows_s[rr], gs_s[rr])

        def gather_wait(ii, rr):
            pltpu.make_async_copy(
                xp_hbm.at[src_s[ii]], rows_s[rr], gs_s[rr]).wait()

        def scatter_start(ii, rr):
            pltpu.async_copy(rows_s[rr], acc.at[dst_s[ii]], ss_s[rr],
                             add=True)

        def scatter_wait(ii, rr):
            pltpu.make_async_copy(
                rows_s[rr], acc.at[dst_s[ii]], ss_s[rr]).wait()

        idx_start(0, 0)
        idx_wait(0, 0)
        gather_start(0, 0)
        idx_start(1, 1)

        def body(t, carry):
            c0 = 4 * t
            idx_start(c0 + 2, 2)

            @pl.when(t > 0)
            def _():
                scatter_wait(3, 1)
            idx_start(c0 + 3, 3)
            gather_wait(0, 0)
            scatter_start(0, 0)
            deg_update(dst_s[0])
            idx_wait(c0 + 1, 1)
            gather_start(1, 1)
            scatter_wait(0, 0)
            idx_start(c0 + 4, 0)
            gather_wait(1, 1)
            scatter_start(1, 1)
            deg_update(dst_s[1])
            idx_wait(c0 + 2, 2)
            gather_start(2, 0)
            scatter_wait(1, 1)
            idx_start(c0 + 5, 1)
            gather_wait(2, 0)
            scatter_start(2, 0)
            deg_update(dst_s[2])
            idx_wait(c0 + 3, 3)
            gather_start(3, 1)
            scatter_wait(2, 0)
            gather_wait(3, 1)
            scatter_start(3, 1)
            deg_update(dst_s[3])
            idx_wait(c0 + 4, 0)
            gather_start(0, 0)
            return carry
        lax.fori_loop(0, BASE_CH // 4, body, 0)

        scatter_wait(3, 1)
        gather_wait(0, 0)
        scatter_start(0, 0)
        deg_update(dst_s[0])
        idx_wait(77, 1)
        gather_start(1, 1)
        scatter_wait(0, 0)

        @pl.when(nch > BASE_CH)
        def _():
            idx_start(78, 2)
            idx_wait(78, 2)
            gather_start(2, 0)
        gather_wait(1, 1)
        scatter_start(1, 1)
        deg_update(dst_s[1])
        scatter_wait(1, 1)

        @pl.when(nch > BASE_CH)
        def _():
            gather_wait(2, 0)
            pltpu.sync_copy(rows_s[0], acc.at[dst_s[2]], add=True)
            deg_update(dst_s[2])

        if with_deg:
            pltpu.sync_copy(deg_v, deg_acc.at[iota_v], add=True)
        plsc.subcore_barrier()

        @pl.when(sid < 15)
        def _():
            pltpu.sync_copy(acc.at[pl.ds(r0, 624)],
                            out_hbm.at[cid, pl.ds(r0, 624)])

        @pl.when(sid == 15)
        def _():
            pltpu.sync_copy(acc.at[pl.ds(9360, 640)],
                            out_hbm.at[cid, pl.ds(9360, 640)])
        if with_deg:
            @pl.when(sid < 10)
            def _():
                dr = pl.multiple_of(sid * 8, 8)
                pltpu.sync_copy(deg_acc.at[pl.ds(dr, 8)],
                                deg_hbm.at[cid, pl.ds(dr, 8)])

    return agg


_sc_agg_deg = _make_sc_agg(True, 2)
_sc_agg_plain = _make_sc_agg(False, 2)


_BM = 2000


def _tc_lin_body(x_ref, w_ref, b_ref, o_ref):
    o_ref[...] = jnp.dot(x_ref[...], w_ref[...],
                         preferred_element_type=jnp.float32) + b_ref[...]


def _tc_lin(xin, wT, b):
    grid = N_NODES // _BM
    return pl.pallas_call(
        _tc_lin_body,
        grid=(grid,),
        in_specs=[
            pl.BlockSpec((_BM, D_FEAT), lambda i: (i, 0)),
            pl.BlockSpec((D_FEAT, D_FEAT), lambda i: (0, 0)),
            pl.BlockSpec((1, D_FEAT), lambda i: (0, 0)),
        ],
        out_specs=pl.BlockSpec((_BM, D_FEAT), lambda i: (i, 0)),
        out_shape=jax.ShapeDtypeStruct((N_NODES, D_FEAT), jnp.float32),
    )(xin, wT, b)


def _tc_comb_body(relu, parts_ref, deg_ref, xr_ref, wl_ref, o_ref):
    p = parts_ref[...]
    s = p[0] + p[1]
    d = deg_ref[...]
    deg = jnp.maximum(d[0] + d[1], 1.0)
    mean = s / deg
    acc = jnp.dot(mean, wl_ref[...], preferred_element_type=jnp.float32)
    acc = acc + xr_ref[...]
    o_ref[...] = jnp.maximum(acc, 0.0) if relu else acc


def _tc_comb(relu, parts, deg3, xr, wlT):
    grid = N_NODES // _BM
    return pl.pallas_call(
        functools.partial(_tc_comb_body, relu),
        grid=(grid,),
        in_specs=[
            pl.BlockSpec((2, _BM, D_FEAT), lambda i: (0, i, 0)),
            pl.BlockSpec((2, _BM, 1), lambda i: (0, i, 0)),
            pl.BlockSpec((_BM, D_FEAT), lambda i: (i, 0)),
            pl.BlockSpec((D_FEAT, D_FEAT), lambda i: (0, 0)),
        ],
        out_specs=pl.BlockSpec((_BM, D_FEAT), lambda i: (i, 0)),
        out_shape=jax.ShapeDtypeStruct((N_NODES, D_FEAT), jnp.float32),
    )(parts, deg3, xr, wlT)


def kernel(x, edge_index, W1l, b1l, W1r, W2l, b2l, W2r):
    src = edge_index[0].astype(jnp.int32)
    dst = edge_index[1].astype(jnp.int32)

    parts1, degp = _sc_agg_deg(x, src, dst)
    xr1 = _tc_lin(x, W1r.T, b1l.reshape(1, D_FEAT))
    deg3 = degp.reshape(2, NPAD, 1)
    h = _tc_comb(True, parts1, deg3, xr1, W1l.T)
    parts2, = _sc_agg_plain(h, src, dst)
    hr2 = _tc_lin(h, W2r.T, b2l.reshape(1, D_FEAT))
    out = _tc_comb(False, parts2, deg3, hr2, W2l.T)
    return out


# speedup vs baseline: 1.0599x; 1.0599x over previous
import functools

import jax
import jax.numpy as jnp
from jax import lax
from jax.experimental import pallas as pl
from jax.experimental.pallas import tpu as pltpu
from jax.experimental.pallas import tpu_sc as plsc

N_NODES = 10000
N_EDGES = 320000
D_FEAT = 128
NPAD = 10240
NW = 32
CHUNK = 128
NCHUNKS = N_EDGES // CHUNK
BASE_CH = NCHUNKS // NW
EXTRA = NCHUNKS - BASE_CH * NW
NPT = NPAD // 16
DROWS = NPAD // 128


def _make_sc_agg(with_deg, NSLOT):
    mesh = plsc.VectorSubcoreMesh(core_axis_name="c", subcore_axis_name="s")
    out_type = [jax.ShapeDtypeStruct((2, N_NODES, D_FEAT), jnp.float32)]
    scratch = [
        [pltpu.VMEM((CHUNK,), jnp.int32) for _ in range(NSLOT)],
        [pltpu.VMEM((CHUNK,), jnp.int32) for _ in range(NSLOT)],
        [pltpu.VMEM((CHUNK, D_FEAT), jnp.float32) for _ in range(NSLOT)],
        pltpu.VMEM_SHARED((N_NODES, D_FEAT), jnp.float32),
        [pltpu.SemaphoreType.DMA for _ in range(NSLOT)],
        [pltpu.SemaphoreType.DMA for _ in range(NSLOT)],
        [pltpu.SemaphoreType.DMA for _ in range(NSLOT)],
    ]
    if with_deg:
        out_type.append(jax.ShapeDtypeStruct((2, DROWS, D_FEAT), jnp.float32))
        scratch += [
            pltpu.VMEM((DROWS, D_FEAT), jnp.float32),
            pltpu.VMEM((DROWS,), jnp.int32),
            pltpu.VMEM_SHARED((DROWS, D_FEAT), jnp.float32),
        ]

    @functools.partial(
        pl.kernel, mesh=mesh, out_type=out_type, scratch_types=scratch,
        compiler_params=pltpu.CompilerParams(needs_layout_passes=False))
    def agg(xp_hbm, src_hbm, dst_hbm, *rest):
        if with_deg:
            (out_hbm, deg_hbm, src_s, dst_s, rows_s, acc,
             is_s, gs_s, ss_s, deg_v, iota_v, deg_acc) = rest
        else:
            (out_hbm, src_s, dst_s, rows_s, acc, is_s, gs_s, ss_s) = rest
        rows_v = rows_s[0]
        cid = lax.axis_index("c")
        sid = lax.axis_index("s")
        wid = sid * 2 + cid

        def zrow(i, carry):
            for j in range(D_FEAT // 16):
                rows_v[i, pl.ds(j * 16, 16)] = jnp.zeros((16,), jnp.float32)
            return carry
        lax.fori_loop(0, CHUNK, zrow, 0)
        r0 = pl.multiple_of(sid * 624, 8)

        @pl.when(sid < 15)
        def _():
            for m in range(4):
                pltpu.sync_copy(rows_v.at[pl.ds(0, CHUNK)],
                                acc.at[pl.ds(r0 + m * CHUNK, CHUNK)])
            pltpu.sync_copy(rows_v.at[pl.ds(0, 112)],
                            acc.at[pl.ds(r0 + 512, 112)])

        @pl.when(sid == 15)
        def _():
            for m in range(5):
                pltpu.sync_copy(rows_v.at[pl.ds(0, CHUNK)],
                                acc.at[pl.ds(9360 + m * CHUNK, CHUNK)])
        if with_deg:
            def zdeg(i, carry):
                for j in range(D_FEAT // 16):
                    deg_v[i, pl.ds(j * 16, 16)] = jnp.zeros((16,), jnp.float32)
                return carry
            lax.fori_loop(0, DROWS, zdeg, 0)
            for m in range(DROWS // 16):
                iota_v[pl.ds(m * 16, 16)] = (
                    lax.iota(jnp.int32, 16) + m * 16)

            @pl.when(sid == 0)
            def _():
                pltpu.sync_copy(rows_v.at[pl.ds(0, DROWS)], deg_acc)
        plsc.subcore_barrier()

        cstart = wid * BASE_CH + jnp.minimum(wid, EXTRA)
        nch = jnp.where(wid < EXTRA, BASE_CH + 1, BASE_CH)

        def idx_slices(c):
            eb = pl.multiple_of((cstart + c) * CHUNK, CHUNK)
            return src_hbm.at[pl.ds(eb, CHUNK)], dst_hbm.at[pl.ds(eb, CHUNK)]

        def idx_start(c, s):
            sr, dr = idx_slices(c)
            pltpu.async_copy(sr, src_s[s], is_s[s])
            pltpu.async_copy(dr, dst_s[s], is_s[s])

        def idx_wait(c, s):
            sr, dr = idx_slices(c)
            pltpu.make_async_copy(sr, src_s[s], is_s[s]).wait()
            pltpu.make_async_copy(dr, dst_s[s], is_s[s]).wait()

        def deg_update(dstx):
            if with_deg:
                for j in range(CHUNK // 16):
                    d16 = dstx[pl.ds(j * 16, 16)]
                    cnt, mlast = plsc.scan_count(d16)
                    plsc.addupdate_scatter(
                        deg_v,
                        [lax.shift_right_logical(d16, 7),
                         jnp.bitwise_and(d16, 127)],
                        cnt.astype(jnp.float32), mask=mlast)

        srcA, srcB = src_s[0], src_s[1]
        dstA, dstB = dst_s[0], dst_s[1]
        rowsA, rowsB = rows_s[0], rows_s[1]
        isA, isB = is_s[0], is_s[1]
        gsA, gsB = gs_s[0], gs_s[1]
        ssA, ssB = ss_s[0], ss_s[1]

        idx_start(0, 0)
        idx_wait(0, 0)
        pltpu.async_copy(xp_hbm.at[srcA], rowsA, gsA)

        def body(i, carry):
            a = 2 * i

            @pl.when(i > 0)
            def _():
                pltpu.make_async_copy(rowsB, acc.at[dstB], ssB).wait()
            idx_start(a + 1, 1)
            pltpu.make_async_copy(xp_hbm.at[srcA], rowsA, gsA).wait()
            pltpu.async_copy(rowsA, acc.at[dstA], ssA, add=True)
            deg_update(dstA)
            idx_wait(a + 1, 1)
            pltpu.async_copy(xp_hbm.at[srcB], rowsB, gsB)
            pltpu.make_async_copy(rowsA, acc.at[dstA], ssA).wait()

            @pl.when(a + 2 < nch)
            def _(a=a):
                idx_start(a + 2, 0)
                idx_wait(a + 2, 0)
                pltpu.async_copy(xp_hbm.at[srcA], rowsA, gsA)
            pltpu.make_async_copy(xp_hbm.at[srcB], rowsB, gsB).wait()
            pltpu.async_copy(rowsB, acc.at[dstB], ssB, add=True)
            deg_update(dstB)
            return carry
        lax.fori_loop(0, BASE_CH // 2, body, 0)

        pltpu.make_async_copy(rowsB, acc.at[dstB], ssB).wait()

        @pl.when(nch > BASE_CH)
        def _():
            pltpu.make_async_copy(xp_hbm.at[srcA], rowsA, gsA).wait()
            pltpu.sync_copy(rowsA, acc.at[dstA], add=True)
            deg_update(dstA)

        if with_deg:
            pltpu.sync_copy(deg_v, deg_acc.at[iota_v], add=True)
        plsc.subcore_barrier()

        @pl.when(sid < 15)
        def _():
            pltpu.sync_copy(acc.at[pl.ds(r0, 624)],
                            out_hbm.at[cid, pl.ds(r0, 624)])

        @pl.when(sid == 15)
        def _():
            pltpu.sync_copy(acc.at[pl.ds(9360, 640)],
                            out_hbm.at[cid, pl.ds(9360, 640)])
        if with_deg:
            @pl.when(sid < 10)
            def _():
                dr = pl.multiple_of(sid * 8, 8)
                pltpu.sync_copy(deg_acc.at[pl.ds(dr, 8)],
                                deg_hbm.at[cid, pl.ds(dr, 8)])

    return agg


_sc_agg_deg = _make_sc_agg(True, 2)
_sc_agg_plain = _make_sc_agg(False, 2)


_BM = 2000


def _tc1_body(parts_ref, deg_ref, x_ref, wl_ref, bl_ref, wr_ref, o_ref):
    p = parts_ref[...]
    s = p[0] + p[1]
    d = deg_ref[...]
    deg = jnp.maximum(d[0] + d[1], 1.0)
    mean = s / deg
    acc = jnp.dot(mean, wl_ref[...], preferred_element_type=jnp.float32)
    acc = acc + jnp.dot(x_ref[...], wr_ref[...],
                        preferred_element_type=jnp.float32)
    acc = acc + bl_ref[...]
    o_ref[...] = jnp.maximum(acc, 0.0)


def _tc2_body(parts_ref, deg_ref, h_ref, wl_ref, bl_ref, wr_ref, o_ref):
    p = parts_ref[...]
    s = p[0] + p[1]
    d = deg_ref[...]
    deg = jnp.maximum(d[0] + d[1], 1.0)
    mean = s / deg
    acc = jnp.dot(mean, wl_ref[...], preferred_element_type=jnp.float32)
    acc = acc + jnp.dot(h_ref[...], wr_ref[...],
                        preferred_element_type=jnp.float32)
    o_ref[...] = acc + bl_ref[...]


def _tc_layer(body, parts, deg3, xin, wlT, bl, wrT):
    grid = N_NODES // _BM
    return pl.pallas_call(
        body,
        grid=(grid,),
        in_specs=[
            pl.BlockSpec((2, _BM, D_FEAT), lambda i: (0, i, 0)),
            pl.BlockSpec((2, _BM, 1), lambda i: (0, i, 0)),
            pl.BlockSpec((_BM, D_FEAT), lambda i: (i, 0)),
            pl.BlockSpec((D_FEAT, D_FEAT), lambda i: (0, 0)),
            pl.BlockSpec((1, D_FEAT), lambda i: (0, 0)),
            pl.BlockSpec((D_FEAT, D_FEAT), lambda i: (0, 0)),
        ],
        out_specs=pl.BlockSpec((_BM, D_FEAT), lambda i: (i, 0)),
        out_shape=jax.ShapeDtypeStruct((N_NODES, D_FEAT), jnp.float32),
    )(parts, deg3, xin, wlT, bl, wrT)


def kernel(x, edge_index, W1l, b1l, W1r, W2l, b2l, W2r):
    src = edge_index[0].astype(jnp.int32)
    dst = edge_index[1].astype(jnp.int32)

    parts1, degp = _sc_agg_deg(x, src, dst)
    deg3 = degp.reshape(2, NPAD, 1)
    h = _tc_layer(_tc1_body, parts1, deg3, x,
                  W1l.T, b1l.reshape(1, D_FEAT), W1r.T)
    parts2, = _sc_agg_plain(h, src, dst)
    out = _tc_layer(_tc2_body, parts2, deg3, h,
                    W2l.T, b2l.reshape(1, D_FEAT), W2r.T)
    return out


# speedup vs baseline: 1.0603x; 1.0004x over previous
import functools

import jax
import jax.numpy as jnp
from jax import lax
from jax.experimental import pallas as pl
from jax.experimental.pallas import tpu as pltpu
from jax.experimental.pallas import tpu_sc as plsc

N_NODES = 10000
N_EDGES = 320000
D_FEAT = 128
NPAD = 10240
NW = 32
CHUNK = 128
NCHUNKS = N_EDGES // CHUNK
BASE_CH = NCHUNKS // NW
EXTRA = NCHUNKS - BASE_CH * NW
NPT = NPAD // 16
DROWS = NPAD // 128


def _make_sc_agg(with_deg, NSLOT):
    mesh = plsc.VectorSubcoreMesh(core_axis_name="c", subcore_axis_name="s")
    out_type = [jax.ShapeDtypeStruct((2, N_NODES, D_FEAT), jnp.float32)]
    scratch = [
        [pltpu.VMEM((CHUNK,), jnp.int32) for _ in range(NSLOT)],
        [pltpu.VMEM((CHUNK,), jnp.int32) for _ in range(NSLOT)],
        [pltpu.VMEM((CHUNK, D_FEAT), jnp.float32) for _ in range(NSLOT)],
        pltpu.VMEM_SHARED((N_NODES, D_FEAT), jnp.float32),
        [pltpu.SemaphoreType.DMA for _ in range(NSLOT)],
        [pltpu.SemaphoreType.DMA for _ in range(NSLOT)],
        [pltpu.SemaphoreType.DMA for _ in range(NSLOT)],
    ]
    if with_deg:
        out_type.append(jax.ShapeDtypeStruct((2, DROWS, D_FEAT), jnp.float32))
        scratch += [
            pltpu.VMEM((DROWS, D_FEAT), jnp.float32),
            pltpu.VMEM((DROWS,), jnp.int32),
            pltpu.VMEM_SHARED((DROWS, D_FEAT), jnp.float32),
        ]

    @functools.partial(
        pl.kernel, mesh=mesh, out_type=out_type, scratch_types=scratch,
        compiler_params=pltpu.CompilerParams(needs_layout_passes=False,
                                             skip_device_barrier=True))
    def agg(xp_hbm, src_hbm, dst_hbm, *rest):
        if with_deg:
            (out_hbm, deg_hbm, src_s, dst_s, rows_s, acc,
             is_s, gs_s, ss_s, deg_v, iota_v, deg_acc) = rest
        else:
            (out_hbm, src_s, dst_s, rows_s, acc, is_s, gs_s, ss_s) = rest
        rows_v = rows_s[0]
        cid = lax.axis_index("c")
        sid = lax.axis_index("s")
        wid = sid * 2 + cid

        def zrow(i, carry):
            for j in range(D_FEAT // 16):
                rows_v[i, pl.ds(j * 16, 16)] = jnp.zeros((16,), jnp.float32)
            return carry
        lax.fori_loop(0, CHUNK, zrow, 0)
        r0 = pl.multiple_of(sid * 624, 8)

        @pl.when(sid < 15)
        def _():
            for m in range(4):
                pltpu.sync_copy(rows_v.at[pl.ds(0, CHUNK)],
                                acc.at[pl.ds(r0 + m * CHUNK, CHUNK)])
            pltpu.sync_copy(rows_v.at[pl.ds(0, 112)],
                            acc.at[pl.ds(r0 + 512, 112)])

        @pl.when(sid == 15)
        def _():
            for m in range(5):
                pltpu.sync_copy(rows_v.at[pl.ds(0, CHUNK)],
                                acc.at[pl.ds(9360 + m * CHUNK, CHUNK)])
        if with_deg:
            def zdeg(i, carry):
                for j in range(D_FEAT // 16):
                    deg_v[i, pl.ds(j * 16, 16)] = jnp.zeros((16,), jnp.float32)
                return carry
            lax.fori_loop(0, DROWS, zdeg, 0)
            for m in range(DROWS // 16):
                iota_v[pl.ds(m * 16, 16)] = (
                    lax.iota(jnp.int32, 16) + m * 16)

            @pl.when(sid == 0)
            def _():
                pltpu.sync_copy(rows_v.at[pl.ds(0, DROWS)], deg_acc)
        plsc.subcore_barrier()

        cstart = wid * BASE_CH + jnp.minimum(wid, EXTRA)
        nch = jnp.where(wid < EXTRA, BASE_CH + 1, BASE_CH)

        def idx_slices(c):
            eb = pl.multiple_of((cstart + c) * CHUNK, CHUNK)
            return src_hbm.at[pl.ds(eb, CHUNK)], dst_hbm.at[pl.ds(eb, CHUNK)]

        def idx_start(c, s):
            sr, dr = idx_slices(c)
            pltpu.async_copy(sr, src_s[s], is_s[s])
            pltpu.async_copy(dr, dst_s[s], is_s[s])

        def idx_wait(c, s):
            sr, dr = idx_slices(c)
            pltpu.make_async_copy(sr, src_s[s], is_s[s]).wait()
            pltpu.make_async_copy(dr, dst_s[s], is_s[s]).wait()

        def deg_update(dstx):
            if with_deg:
                for j in range(CHUNK // 16):
                    d16 = dstx[pl.ds(j * 16, 16)]
                    cnt, mlast = plsc.scan_count(d16)
                    plsc.addupdate_scatter(
                        deg_v,
                        [lax.shift_right_logical(d16, 7),
                         jnp.bitwise_and(d16, 127)],
                        cnt.astype(jnp.float32), mask=mlast)

        srcA, srcB = src_s[0], src_s[1]
        dstA, dstB = dst_s[0], dst_s[1]
        rowsA, rowsB = rows_s[0], rows_s[1]
        isA, isB = is_s[0], is_s[1]
        gsA, gsB = gs_s[0], gs_s[1]
        ssA, ssB = ss_s[0], ss_s[1]

        idx_start(0, 0)
        idx_wait(0, 0)
        pltpu.async_copy(xp_hbm.at[srcA], rowsA, gsA)

        def body(i, carry):
            a = 2 * i

            @pl.when(i > 0)
            def _():
                pltpu.make_async_copy(rowsB, acc.at[dstB], ssB).wait()
            idx_start(a + 1, 1)
            pltpu.make_async_copy(xp_hbm.at[srcA], rowsA, gsA).wait()
            pltpu.async_copy(rowsA, acc.at[dstA], ssA, add=True)
            deg_update(dstA)
            idx_wait(a + 1, 1)
            pltpu.async_copy(xp_hbm.at[srcB], rowsB, gsB)
            pltpu.make_async_copy(rowsA, acc.at[dstA], ssA).wait()

            @pl.when(a + 2 < nch)
            def _(a=a):
                idx_start(a + 2, 0)
                idx_wait(a + 2, 0)
                pltpu.async_copy(xp_hbm.at[srcA], rowsA, gsA)
            pltpu.make_async_copy(xp_hbm.at[srcB], rowsB, gsB).wait()
            pltpu.async_copy(rowsB, acc.at[dstB], ssB, add=True)
            deg_update(dstB)
            return carry
        lax.fori_loop(0, BASE_CH // 2, body, 0)

        pltpu.make_async_copy(rowsB, acc.at[dstB], ssB).wait()

        @pl.when(nch > BASE_CH)
        def _():
            pltpu.make_async_copy(xp_hbm.at[srcA], rowsA, gsA).wait()
            pltpu.sync_copy(rowsA, acc.at[dstA], add=True)
            deg_update(dstA)

        if with_deg:
            pltpu.sync_copy(deg_v, deg_acc.at[iota_v], add=True)
        plsc.subcore_barrier()

        @pl.when(sid < 15)
        def _():
            pltpu.sync_copy(acc.at[pl.ds(r0, 624)],
                            out_hbm.at[cid, pl.ds(r0, 624)])

        @pl.when(sid == 15)
        def _():
            pltpu.sync_copy(acc.at[pl.ds(9360, 640)],
                            out_hbm.at[cid, pl.ds(9360, 640)])
        if with_deg:
            @pl.when(sid < 10)
            def _():
                dr = pl.multiple_of(sid * 8, 8)
                pltpu.sync_copy(deg_acc.at[pl.ds(dr, 8)],
                                deg_hbm.at[cid, pl.ds(dr, 8)])

    return agg


_sc_agg_deg = _make_sc_agg(True, 2)
_sc_agg_plain = _make_sc_agg(False, 2)


_BM = 2000


def _tc1_body(parts_ref, deg_ref, x_ref, wl_ref, bl_ref, wr_ref, o_ref):
    p = parts_ref[...]
    s = p[0] + p[1]
    d = deg_ref[...]
    deg = jnp.maximum(d[0] + d[1], 1.0)
    mean = s / deg
    acc = jnp.dot(mean, wl_ref[...], preferred_element_type=jnp.float32)
    acc = acc + jnp.dot(x_ref[...], wr_ref[...],
                        preferred_element_type=jnp.float32)
    acc = acc + bl_ref[...]
    o_ref[...] = jnp.maximum(acc, 0.0)


def _tc2_body(parts_ref, deg_ref, h_ref, wl_ref, bl_ref, wr_ref, o_ref):
    p = parts_ref[...]
    s = p[0] + p[1]
    d = deg_ref[...]
    deg = jnp.maximum(d[0] + d[1], 1.0)
    mean = s / deg
    acc = jnp.dot(mean, wl_ref[...], preferred_element_type=jnp.float32)
    acc = acc + jnp.dot(h_ref[...], wr_ref[...],
                        preferred_element_type=jnp.float32)
    o_ref[...] = acc + bl_ref[...]


def _tc_layer(body, parts, deg3, xin, wlT, bl, wrT):
    grid = N_NODES // _BM
    return pl.pallas_call(
        body,
        grid=(grid,),
        in_specs=[
            pl.BlockSpec((2, _BM, D_FEAT), lambda i: (0, i, 0)),
            pl.BlockSpec((2, _BM, 1), lambda i: (0, i, 0)),
            pl.BlockSpec((_BM, D_FEAT), lambda i: (i, 0)),
            pl.BlockSpec((D_FEAT, D_FEAT), lambda i: (0, 0)),
            pl.BlockSpec((1, D_FEAT), lambda i: (0, 0)),
            pl.BlockSpec((D_FEAT, D_FEAT), lambda i: (0, 0)),
        ],
        out_specs=pl.BlockSpec((_BM, D_FEAT), lambda i: (i, 0)),
        out_shape=jax.ShapeDtypeStruct((N_NODES, D_FEAT), jnp.float32),
    )(parts, deg3, xin, wlT, bl, wrT)


def kernel(x, edge_index, W1l, b1l, W1r, W2l, b2l, W2r):
    src = edge_index[0].astype(jnp.int32)
    dst = edge_index[1].astype(jnp.int32)

    parts1, degp = _sc_agg_deg(x, src, dst)
    deg3 = degp.reshape(2, NPAD, 1)
    h = _tc_layer(_tc1_body, parts1, deg3, x,
                  W1l.T, b1l.reshape(1, D_FEAT), W1r.T)
    parts2, = _sc_agg_plain(h, src, dst)
    out = _tc_layer(_tc2_body, parts2, deg3, h,
                    W2l.T, b2l.reshape(1, D_FEAT), W2r.T)
    return out


# speedup vs baseline: 1.1067x; 1.0438x over previous
import functools

import jax
import jax.numpy as jnp
from jax import lax
from jax.experimental import pallas as pl
from jax.experimental.pallas import tpu as pltpu
from jax.experimental.pallas import tpu_sc as plsc

N_NODES = 10000
N_EDGES = 320000
D_FEAT = 128
NPAD = 10240
NW = 32
CHUNK = 128
NCHUNKS = N_EDGES // CHUNK
BASE_CH = NCHUNKS // NW
EXTRA = NCHUNKS - BASE_CH * NW
NPT = NPAD // 16
DROWS = NPAD // 128


def _make_sc_agg(with_deg, NSLOT):
    mesh = plsc.VectorSubcoreMesh(core_axis_name="c", subcore_axis_name="s")
    out_type = [jax.ShapeDtypeStruct((2, N_NODES, D_FEAT), jnp.float32)]
    scratch = [
        [pltpu.VMEM((CHUNK,), jnp.int32) for _ in range(NSLOT)],
        [pltpu.VMEM((CHUNK,), jnp.int32) for _ in range(NSLOT)],
        [pltpu.VMEM((CHUNK, D_FEAT), jnp.float32) for _ in range(NSLOT)],
        pltpu.VMEM_SHARED((N_NODES, D_FEAT), jnp.float32),
        [pltpu.SemaphoreType.DMA for _ in range(NSLOT)],
        [pltpu.SemaphoreType.DMA for _ in range(NSLOT)],
        [pltpu.SemaphoreType.DMA for _ in range(NSLOT)],
    ]
    if with_deg:
        out_type.append(jax.ShapeDtypeStruct((2, DROWS, D_FEAT), jnp.float32))
        scratch += [
            pltpu.VMEM((DROWS, D_FEAT), jnp.float32),
            pltpu.VMEM((DROWS,), jnp.int32),
            pltpu.VMEM_SHARED((DROWS, D_FEAT), jnp.float32),
        ]

    @functools.partial(
        pl.kernel, mesh=mesh, out_type=out_type, scratch_types=scratch,
        compiler_params=pltpu.CompilerParams(needs_layout_passes=False))
    def agg(xp_hbm, edge_hbm, *rest):
        if with_deg:
            (out_hbm, deg_hbm, src_s, dst_s, rows_s, acc,
             is_s, gs_s, ss_s, deg_v, iota_v, deg_acc) = rest
        else:
            (out_hbm, src_s, dst_s, rows_s, acc, is_s, gs_s, ss_s) = rest
        rows_v = rows_s[0]
        cid = lax.axis_index("c")
        sid = lax.axis_index("s")
        wid = sid * 2 + cid

        def zrow(i, carry):
            for j in range(D_FEAT // 16):
                rows_v[i, pl.ds(j * 16, 16)] = jnp.zeros((16,), jnp.float32)
            return carry
        lax.fori_loop(0, CHUNK, zrow, 0)
        r0 = pl.multiple_of(sid * 624, 8)

        @pl.when(sid < 15)
        def _():
            for m in range(4):
                pltpu.sync_copy(rows_v.at[pl.ds(0, CHUNK)],
                                acc.at[pl.ds(r0 + m * CHUNK, CHUNK)])
            pltpu.sync_copy(rows_v.at[pl.ds(0, 112)],
                            acc.at[pl.ds(r0 + 512, 112)])

        @pl.when(sid == 15)
        def _():
            for m in range(5):
                pltpu.sync_copy(rows_v.at[pl.ds(0, CHUNK)],
                                acc.at[pl.ds(9360 + m * CHUNK, CHUNK)])
        if with_deg:
            def zdeg(i, carry):
                for j in range(D_FEAT // 16):
                    deg_v[i, pl.ds(j * 16, 16)] = jnp.zeros((16,), jnp.float32)
                return carry
            lax.fori_loop(0, DROWS, zdeg, 0)
            for m in range(DROWS // 16):
                iota_v[pl.ds(m * 16, 16)] = (
                    lax.iota(jnp.int32, 16) + m * 16)

            @pl.when(sid == 0)
            def _():
                pltpu.sync_copy(rows_v.at[pl.ds(0, DROWS)], deg_acc)
        plsc.subcore_barrier()

        cstart = wid * BASE_CH + jnp.minimum(wid, EXTRA)
        nch = jnp.where(wid < EXTRA, BASE_CH + 1, BASE_CH)

        def idx_slices(c):
            eb = pl.multiple_of((cstart + c) * CHUNK, CHUNK)
            return (edge_hbm.at[0, pl.ds(eb, CHUNK)],
                    edge_hbm.at[1, pl.ds(eb, CHUNK)])

        def idx_start(c, s):
            sr, dr = idx_slices(c)
            pltpu.async_copy(sr, src_s[s], is_s[s])
            pltpu.async_copy(dr, dst_s[s], is_s[s])

        def idx_wait(c, s):
            sr, dr = idx_slices(c)
            pltpu.make_async_copy(sr, src_s[s], is_s[s]).wait()
            pltpu.make_async_copy(dr, dst_s[s], is_s[s]).wait()

        def deg_update(dstx):
            if with_deg:
                for j in range(CHUNK // 16):
                    d16 = dstx[pl.ds(j * 16, 16)]
                    cnt, mlast = plsc.scan_count(d16)
                    plsc.addupdate_scatter(
                        deg_v,
                        [lax.shift_right_logical(d16, 7),
                         jnp.bitwise_and(d16, 127)],
                        cnt.astype(jnp.float32), mask=mlast)

        srcA, srcB = src_s[0], src_s[1]
        dstA, dstB = dst_s[0], dst_s[1]
        rowsA, rowsB = rows_s[0], rows_s[1]
        isA, isB = is_s[0], is_s[1]
        gsA, gsB = gs_s[0], gs_s[1]
        ssA, ssB = ss_s[0], ss_s[1]

        idx_start(0, 0)
        idx_wait(0, 0)
        pltpu.async_copy(xp_hbm.at[srcA], rowsA, gsA)

        def body(i, carry):
            a = 2 * i

            @pl.when(i > 0)
            def _():
                pltpu.make_async_copy(rowsB, acc.at[dstB], ssB).wait()
            idx_start(a + 1, 1)
            pltpu.make_async_copy(xp_hbm.at[srcA], rowsA, gsA).wait()
            pltpu.async_copy(rowsA, acc.at[dstA], ssA, add=True)
            deg_update(dstA)
            idx_wait(a + 1, 1)
            pltpu.async_copy(xp_hbm.at[srcB], rowsB, gsB)
            pltpu.make_async_copy(rowsA, acc.at[dstA], ssA).wait()

            @pl.when(a + 2 < nch)
            def _(a=a):
                idx_start(a + 2, 0)
                idx_wait(a + 2, 0)
                pltpu.async_copy(xp_hbm.at[srcA], rowsA, gsA)
            pltpu.make_async_copy(xp_hbm.at[srcB], rowsB, gsB).wait()
            pltpu.async_copy(rowsB, acc.at[dstB], ssB, add=True)
            deg_update(dstB)
            return carry
        lax.fori_loop(0, BASE_CH // 2, body, 0)

        pltpu.make_async_copy(rowsB, acc.at[dstB], ssB).wait()

        @pl.when(nch > BASE_CH)
        def _():
            pltpu.make_async_copy(xp_hbm.at[srcA], rowsA, gsA).wait()
            pltpu.sync_copy(rowsA, acc.at[dstA], add=True)
            deg_update(dstA)

        if with_deg:
            pltpu.sync_copy(deg_v, deg_acc.at[iota_v], add=True)
        plsc.subcore_barrier()

        @pl.when(sid < 15)
        def _():
            pltpu.sync_copy(acc.at[pl.ds(r0, 624)],
                            out_hbm.at[cid, pl.ds(r0, 624)])

        @pl.when(sid == 15)
        def _():
            pltpu.sync_copy(acc.at[pl.ds(9360, 640)],
                            out_hbm.at[cid, pl.ds(9360, 640)])
        if with_deg:
            @pl.when(sid < 10)
            def _():
                dr = pl.multiple_of(sid * 8, 8)
                pltpu.sync_copy(deg_acc.at[pl.ds(dr, 8)],
                                deg_hbm.at[cid, pl.ds(dr, 8)])

    return agg


_sc_agg_deg = _make_sc_agg(True, 2)
_sc_agg_plain = _make_sc_agg(False, 2)


_BM = 2000


def _tc1_body(parts_ref, deg_ref, x_ref, wl_ref, bl_ref, wr_ref, o_ref):
    p = parts_ref[...]
    s = p[0] + p[1]
    d = deg_ref[...]
    deg = jnp.maximum(d[0] + d[1], 1.0)
    mean = s / deg
    acc = jnp.dot(mean, wl_ref[...], preferred_element_type=jnp.float32)
    acc = acc + jnp.dot(x_ref[...], wr_ref[...],
                        preferred_element_type=jnp.float32)
    acc = acc + bl_ref[...]
    o_ref[...] = jnp.maximum(acc, 0.0)


def _tc2_body(parts_ref, deg_ref, h_ref, wl_ref, bl_ref, wr_ref, o_ref):
    p = parts_ref[...]
    s = p[0] + p[1]
    d = deg_ref[...]
    deg = jnp.maximum(d[0] + d[1], 1.0)
    mean = s / deg
    acc = jnp.dot(mean, wl_ref[...], preferred_element_type=jnp.float32)
    acc = acc + jnp.dot(h_ref[...], wr_ref[...],
                        preferred_element_type=jnp.float32)
    o_ref[...] = acc + bl_ref[...]


def _tc_layer(body, parts, deg3, xin, wlT, bl, wrT):
    grid = N_NODES // _BM
    return pl.pallas_call(
        body,
        grid=(grid,),
        in_specs=[
            pl.BlockSpec((2, _BM, D_FEAT), lambda i: (0, i, 0)),
            pl.BlockSpec((2, _BM, 1), lambda i: (0, i, 0)),
            pl.BlockSpec((_BM, D_FEAT), lambda i: (i, 0)),
            pl.BlockSpec((D_FEAT, D_FEAT), lambda i: (0, 0)),
            pl.BlockSpec((1, D_FEAT), lambda i: (0, 0)),
            pl.BlockSpec((D_FEAT, D_FEAT), lambda i: (0, 0)),
        ],
        out_specs=pl.BlockSpec((_BM, D_FEAT), lambda i: (i, 0)),
        out_shape=jax.ShapeDtypeStruct((N_NODES, D_FEAT), jnp.float32),
    )(parts, deg3, xin, wlT, bl, wrT)


def kernel(x, edge_index, W1l, b1l, W1r, W2l, b2l, W2r):
    edges = edge_index.astype(jnp.int32)

    parts1, degp = _sc_agg_deg(x, edges)
    deg3 = degp.reshape(2, NPAD, 1)
    h = _tc_layer(_tc1_body, parts1, deg3, x,
                  W1l.T, b1l.reshape(1, D_FEAT), W1r.T)
    parts2, = _sc_agg_plain(h, edges)
    out = _tc_layer(_tc2_body, parts2, deg3, h,
                    W2l.T, b2l.reshape(1, D_FEAT), W2r.T)
    return out
